# Initial kernel scaffold; baseline (speedup 1.0000x reference)
import jax
import jax.numpy as jnp
from jax.experimental import pallas as pl


def kernel(xA, edge_indexA, edge_attrA, batchA, xB, edge_indexB, edge_attrB, batchB, context, params):
    raise NotImplementedError("write your pallas kernel here")


# speedup vs baseline: 2.3105x; 2.3105x over previous
import functools

import jax
import jax.numpy as jnp
from jax import lax
from jax.experimental import pallas as pl
from jax.experimental.pallas import tpu as pltpu
from jax.experimental.pallas import tpu_sc as plsc

N = 10000
E = 160000
B = 64
EMB = 128
NUM_LAYER = 5

NW = 32
CHUNK = 128
EPT = 5120
E_PAD = NW * EPT
NCHUNK = EPT // CHUNK
NODE_PAD = 10112
RPT = NODE_PAD // 16
RB = 400
NB = N // RB


def _idx_body(src_ref, ea_ref, out_ref):
    out_ref[...] = ea_ref[...] * N + src_ref[...]


_idx_kernel = pl.pallas_call(
    _idx_body,
    out_shape=jax.ShapeDtypeStruct((E_PAD // 128, 128), jnp.int32),
)


def _embed_body(x_ref, xemb_ref, eemb_ref, h_ref, t_ref):
    xb = x_ref[0, 0, :]
    oh = (xb[:, None] == lax.broadcasted_iota(jnp.int32, (RB, 120), 1))
    h = jnp.dot(oh.astype(jnp.float32), xemb_ref[...],
                preferred_element_type=jnp.float32)
    h_ref[...] = h
    for j in range(6):
        t_ref[j] = jnp.maximum(h + eemb_ref[j, :], 0.0)


_embed_kernel = pl.pallas_call(
    _embed_body,
    grid=(NB,),
    in_specs=[
        pl.BlockSpec((1, 1, RB), lambda b: (b, 0, 0)),
        pl.BlockSpec((120, 128), lambda b: (0, 0)),
        pl.BlockSpec((8, 128), lambda b: (0, 0)),
    ],
    out_specs=[
        pl.BlockSpec((RB, 128), lambda b: (b, 0)),
        pl.BlockSpec((6, RB, 128), lambda b: (0, b, 0)),
    ],
    out_shape=[
        jax.ShapeDtypeStruct((N, 128), jnp.float32),
        jax.ShapeDtypeStruct((6, N, 128), jnp.float32),
    ],
)


def _mlp_body(h_ref, a0_ref, a1_ref, w1_ref, b1_ref, w2_ref, b2_ref,
              u_ref, st_ref):
    z = h_ref[...] + a0_ref[...] + a1_ref[...]
    t = jnp.maximum(
        jnp.dot(z, w1_ref[...], preferred_element_type=jnp.float32)
        + b1_ref[0, :], 0.0)
    u = (jnp.dot(t, w2_ref[...], preferred_element_type=jnp.float32)
         + b2_ref[0, :])
    u_ref[...] = u
    s1 = jnp.sum(u, axis=0, keepdims=True)
    s2 = jnp.sum(u * u, axis=0, keepdims=True)
    st_ref[0] = jnp.concatenate([s1, s2, jnp.zeros((6, 128), jnp.float32)], 0)


_mlp_kernel = pl.pallas_call(
    _mlp_body,
    grid=(NB,),
    in_specs=[
        pl.BlockSpec((RB, 128), lambda b: (b, 0)),
        pl.BlockSpec((RB, 128), lambda b: (b, 0)),
        pl.BlockSpec((RB, 128), lambda b: (b, 0)),
        pl.BlockSpec((128, 256), lambda b: (0, 0)),
        pl.BlockSpec((1, 256), lambda b: (0, 0)),
        pl.BlockSpec((256, 128), lambda b: (0, 0)),
        pl.BlockSpec((1, 128), lambda b: (0, 0)),
    ],
    out_specs=[
        pl.BlockSpec((RB, 128), lambda b: (b, 0)),
        pl.BlockSpec((1, 8, 128), lambda b: (b, 0, 0)),
    ],
    out_shape=[
        jax.ShapeDtypeStruct((N, 128), jnp.float32),
        jax.ShapeDtypeStruct((NB, 8, 128), jnp.float32),
    ],
)


def _bn_scale_shift(st_ref, g_ref, be_ref):
    s = st_ref[...]
    mu = jnp.sum(s[:, 0, :], axis=0) / N
    ex2 = jnp.sum(s[:, 1, :], axis=0) / N
    var = ex2 - mu * mu
    scale = g_ref[0, :] * lax.rsqrt(var + 1e-5)
    shift = be_ref[0, :] - mu * scale
    return scale, shift


def _bn_next_body(u_ref, st_ref, g_ref, be_ref, eemb_ref, h_ref, t_ref):
    scale, shift = _bn_scale_shift(st_ref, g_ref, be_ref)
    hn = jnp.maximum(u_ref[...] * scale + shift, 0.0)
    h_ref[...] = hn
    for j in range(6):
        t_ref[j] = jnp.maximum(hn + eemb_ref[j, :], 0.0)


_bn_next_kernel = pl.pallas_call(
    _bn_next_body,
    grid=(NB,),
    in_specs=[
        pl.BlockSpec((RB, 128), lambda b: (b, 0)),
        pl.BlockSpec((NB, 8, 128), lambda b: (0, 0, 0)),
        pl.BlockSpec((1, 128), lambda b: (0, 0)),
        pl.BlockSpec((1, 128), lambda b: (0, 0)),
        pl.BlockSpec((8, 128), lambda b: (0, 0)),
    ],
    out_specs=[
        pl.BlockSpec((RB, 128), lambda b: (b, 0)),
        pl.BlockSpec((6, RB, 128), lambda b: (0, b, 0)),
    ],
    out_shape=[
        jax.ShapeDtypeStruct((N, 128), jnp.float32),
        jax.ShapeDtypeStruct((6, N, 128), jnp.float32),
    ],
)


def _bn_pool_body(u_ref, st_ref, g_ref, be_ref, batch_ref,
                  pooled_ref, cnt_ref):
    scale, shift = _bn_scale_shift(st_ref, g_ref, be_ref)
    hn = u_ref[...] * scale + shift
    b = pl.program_id(0)

    @pl.when(b == 0)
    def _():
        pooled_ref[...] = jnp.zeros((B, 128), jnp.float32)
        cnt_ref[...] = jnp.zeros((B, 128), jnp.float32)

    bb = batch_ref[0, 0, :]
    mask = (lax.broadcasted_iota(jnp.int32, (B, RB), 0)
            == bb[None, :]).astype(jnp.float32)
    pooled_ref[...] += jnp.dot(mask, hn, preferred_element_type=jnp.float32)
    cnt_ref[...] += jnp.broadcast_to(jnp.sum(mask, axis=1)[:, None], (B, 128))


_bn_pool_kernel = pl.pallas_call(
    _bn_pool_body,
    grid=(NB,),
    in_specs=[
        pl.BlockSpec((RB, 128), lambda b: (b, 0)),
        pl.BlockSpec((NB, 8, 128), lambda b: (0, 0, 0)),
        pl.BlockSpec((1, 128), lambda b: (0, 0)),
        pl.BlockSpec((1, 128), lambda b: (0, 0)),
        pl.BlockSpec((1, 1, RB), lambda b: (b, 0, 0)),
    ],
    out_specs=[
        pl.BlockSpec((B, 128), lambda b: (0, 0)),
        pl.BlockSpec((B, 128), lambda b: (0, 0)),
    ],
    out_shape=[
        jax.ShapeDtypeStruct((B, 128), jnp.float32),
        jax.ShapeDtypeStruct((B, 128), jnp.float32),
    ],
)


def _tail_body(pa_ref, ca_ref, pb_ref, cb_ref, ctx_ref,
               wc1_ref, bc1_ref, wc2_ref, bc2_ref,
               wo1_ref, bo1_ref, wo2_ref, bo2_ref,
               wm1_ref, bm1_ref, wm2_ref, bm2_ref, wm3_ref, bm3_ref,
               out_ref):
    def dense(x, w_ref, b_ref, act):
        y = (jnp.dot(x, w_ref[...], preferred_element_type=jnp.float32)
             + b_ref[0, :])
        return jnp.maximum(y, 0.0) if act else y

    pa = pa_ref[...] / jnp.maximum(ca_ref[...], 1.0)
    pb = pb_ref[...] / jnp.maximum(cb_ref[...], 1.0)
    ha = dense(dense(pa, wo1_ref, bo1_ref, True), wo2_ref, bo2_ref, False)
    hb = dense(dense(pb, wo1_ref, bo1_ref, True), wo2_ref, bo2_ref, False)
    ctx = dense(dense(ctx_ref[...], wc1_ref, bc1_ref, True),
                wc2_ref, bc2_ref, False)
    z = jnp.concatenate([ha, hb, ctx], axis=1)
    z = dense(z, wm1_ref, bm1_ref, True)
    z = dense(z, wm2_ref, bm2_ref, True)
    out_ref[...] = dense(z, wm3_ref, bm3_ref, False)


_tail_kernel = pl.pallas_call(
    _tail_body,
    out_shape=jax.ShapeDtypeStruct((B, 128), jnp.float32),
)


_sc_mesh = plsc.VectorSubcoreMesh(core_axis_name="c", subcore_axis_name="s")


def _sc_aggr_body(t_hbm, comb_hbm, dst_hbm, zeros_hbm, out_hbm,
                  idxv, dstv, buf_a, buf_b, shared, sem_a, sem_b):
    c = lax.axis_index("c")
    s = lax.axis_index("s")
    wid = s * 2 + c

    pltpu.sync_copy(zeros_hbm.at[pl.ds(s * RPT, RPT)],
                    shared.at[pl.ds(s * RPT, RPT)])
    pltpu.sync_copy(comb_hbm.at[wid], idxv)
    pltpu.sync_copy(dst_hbm.at[wid], dstv)
    plsc.subcore_barrier()

    def body(i, carry):
        j0 = 2 * i
        j1 = j0 + 1
        cp_a = pltpu.async_copy(t_hbm.at[idxv.at[j0]], buf_a, sem_a)
        cp_b = pltpu.async_copy(t_hbm.at[idxv.at[j1]], buf_b, sem_b)
        cp_a.wait()
        pltpu.sync_copy(buf_a, shared.at[dstv.at[j0]], add=True)
        cp_b.wait()
        pltpu.sync_copy(buf_b, shared.at[dstv.at[j1]], add=True)
        return carry

    lax.fori_loop(0, NCHUNK // 2, body, jnp.int32(0))
    plsc.subcore_barrier()
    pltpu.sync_copy(shared.at[pl.ds(s * RPT, RPT)],
                    out_hbm.at[c, pl.ds(s * RPT, RPT)])


_sc_aggr = functools.partial(
    pl.kernel,
    out_type=jax.ShapeDtypeStruct((2, NODE_PAD, 128), jnp.float32),
    mesh=_sc_mesh,
    scratch_types=[
        pltpu.VMEM((NCHUNK, CHUNK), jnp.int32),
        pltpu.VMEM((NCHUNK, CHUNK), jnp.int32),
        pltpu.VMEM((CHUNK, 128), jnp.float32),
        pltpu.VMEM((CHUNK, 128), jnp.float32),
        pltpu.VMEM_SHARED((NODE_PAD, 128), jnp.float32),
        pltpu.SemaphoreType.DMA,
        pltpu.SemaphoreType.DMA,
    ],
)(_sc_aggr_body)


def _pad_edges(v, fill):
    return jnp.concatenate(
        [v.astype(jnp.int32), jnp.full((E_PAD - E,), fill, jnp.int32)])


def kernel(xA, edge_indexA, edge_attrA, batchA, xB, edge_indexB, edge_attrB,
           batchB, context, params):
    x_emb = params['x_emb']
    e_emb8 = jnp.concatenate(
        [params['e_emb'], jnp.zeros((2, 128), jnp.float32)], axis=0)
    zeros = jnp.zeros((NODE_PAD, 128), jnp.float32)

    def branch(x, ei, ea, batch):
        src = _pad_edges(ei[0], 0).reshape(E_PAD // 128, 128)
        eav = _pad_edges(ea[:, 0], 0).reshape(E_PAD // 128, 128)
        dst3 = _pad_edges(ei[1], N).reshape(NW, NCHUNK, CHUNK)
        comb3 = _idx_kernel(src, eav).reshape(NW, NCHUNK, CHUNK)
        x3 = x[:, 0].astype(jnp.int32).reshape(NB, 1, RB)
        batch3 = batch.astype(jnp.int32).reshape(NB, 1, RB)

        h, t = _embed_kernel(x3, x_emb, e_emb8)
        for l in range(NUM_LAYER):
            p = params['gnn'][l]
            aggr = _sc_aggr(t.reshape(6 * N, 128), comb3, dst3, zeros)
            u, st = _mlp_kernel(h, aggr[0], aggr[1],
                                p['W1'], p['b1'].reshape(1, 256),
                                p['W2'], p['b2'].reshape(1, 128))
            g = p['gamma'].reshape(1, 128)
            be = p['beta'].reshape(1, 128)
            if l < NUM_LAYER - 1:
                h, t = _bn_next_kernel(u, st, g, be, e_emb8)
            else:
                pooled, cnt = _bn_pool_kernel(u, st, g, be, batch3)
        return pooled, cnt

    pooledA, cntA = branch(xA, edge_indexA, edge_attrA, batchA)
    pooledB, cntB = branch(xB, edge_indexB, edge_attrB, batchB)

    ctx_pad = jnp.pad(context, ((0, 0), (0, 1024 - context.shape[1])))
    wc1_pad = jnp.pad(params['Wc1'], ((0, 1024 - params['Wc1'].shape[0]),
                                      (0, 0)))
    wm3_pad = jnp.pad(params['Wm3'], ((0, 0), (0, 127)))
    bm3_pad = jnp.pad(params['bm3'], ((0, 127),))

    out = _tail_kernel(
        pooledA, cntA, pooledB, cntB, ctx_pad,
        wc1_pad, params['bc1'].reshape(1, 512),
        params['Wc2'], params['bc2'].reshape(1, 128),
        params['Wo1'], params['bo1'].reshape(1, 512),
        params['Wo2'], params['bo2'].reshape(1, 256),
        params['Wm1'], params['bm1'].reshape(1, 256),
        params['Wm2'], params['bm2'].reshape(1, 64),
        wm3_pad, bm3_pad.reshape(1, 128))
    return out[:, :1]


# speedup vs baseline: 2.3293x; 1.0081x over previous
import functools

import jax
import jax.numpy as jnp
from jax import lax
from jax.experimental import pallas as pl
from jax.experimental.pallas import tpu as pltpu
from jax.experimental.pallas import tpu_sc as plsc

N = 10000
E = 160000
B = 64
EMB = 128
NUM_LAYER = 5

NW = 32
CHUNK = 128
EPT = 5120
E_PAD = NW * EPT
NCHUNK = EPT // CHUNK
NODE_PAD = 10112
RPT = NODE_PAD // 16
RB = 400
NB = N // RB


def _idx_body(src_ref, ea_ref, out_ref):
    out_ref[...] = ea_ref[...] * N + src_ref[...]


_idx_kernel = pl.pallas_call(
    _idx_body,
    out_shape=jax.ShapeDtypeStruct((E_PAD // 128, 128), jnp.int32),
)


def _embed_body(x_ref, xemb_ref, eemb_ref, h_ref, t_ref):
    xb = x_ref[0, 0, :]
    oh = (xb[:, None] == lax.broadcasted_iota(jnp.int32, (RB, 120), 1))
    h = jnp.dot(oh.astype(jnp.float32), xemb_ref[...],
                preferred_element_type=jnp.float32)
    h_ref[...] = h
    for j in range(6):
        t_ref[j] = jnp.maximum(h + eemb_ref[j, :], 0.0)


_embed_kernel = pl.pallas_call(
    _embed_body,
    grid=(NB,),
    in_specs=[
        pl.BlockSpec((1, 1, RB), lambda b: (b, 0, 0)),
        pl.BlockSpec((120, 128), lambda b: (0, 0)),
        pl.BlockSpec((8, 128), lambda b: (0, 0)),
    ],
    out_specs=[
        pl.BlockSpec((RB, 128), lambda b: (b, 0)),
        pl.BlockSpec((6, RB, 128), lambda b: (0, b, 0)),
    ],
    out_shape=[
        jax.ShapeDtypeStruct((N, 128), jnp.float32),
        jax.ShapeDtypeStruct((6, N, 128), jnp.float32),
    ],
)


def _mlp_body(h_ref, a0_ref, a1_ref, w1_ref, b1_ref, w2_ref, b2_ref,
              u_ref, st_ref):
    z = h_ref[...] + a0_ref[...] + a1_ref[...]
    t = jnp.maximum(
        jnp.dot(z, w1_ref[...], preferred_element_type=jnp.float32)
        + b1_ref[0, :], 0.0)
    u = (jnp.dot(t, w2_ref[...], preferred_element_type=jnp.float32)
         + b2_ref[0, :])
    u_ref[...] = u
    s1 = jnp.sum(u, axis=0, keepdims=True)
    s2 = jnp.sum(u * u, axis=0, keepdims=True)
    st_ref[0] = jnp.concatenate([s1, s2, jnp.zeros((6, 128), jnp.float32)], 0)


_mlp_kernel = pl.pallas_call(
    _mlp_body,
    grid=(NB,),
    in_specs=[
        pl.BlockSpec((RB, 128), lambda b: (b, 0)),
        pl.BlockSpec((RB, 128), lambda b: (b, 0)),
        pl.BlockSpec((RB, 128), lambda b: (b, 0)),
        pl.BlockSpec((128, 256), lambda b: (0, 0)),
        pl.BlockSpec((1, 256), lambda b: (0, 0)),
        pl.BlockSpec((256, 128), lambda b: (0, 0)),
        pl.BlockSpec((1, 128), lambda b: (0, 0)),
    ],
    out_specs=[
        pl.BlockSpec((RB, 128), lambda b: (b, 0)),
        pl.BlockSpec((1, 8, 128), lambda b: (b, 0, 0)),
    ],
    out_shape=[
        jax.ShapeDtypeStruct((N, 128), jnp.float32),
        jax.ShapeDtypeStruct((NB, 8, 128), jnp.float32),
    ],
)


def _bn_scale_shift(st_ref, g_ref, be_ref):
    s = st_ref[...]
    mu = jnp.sum(s[:, 0, :], axis=0) / N
    ex2 = jnp.sum(s[:, 1, :], axis=0) / N
    var = ex2 - mu * mu
    scale = g_ref[0, :] * lax.rsqrt(var + 1e-5)
    shift = be_ref[0, :] - mu * scale
    return scale, shift


def _bn_next_body(u_ref, st_ref, g_ref, be_ref, eemb_ref, h_ref, t_ref):
    scale, shift = _bn_scale_shift(st_ref, g_ref, be_ref)
    hn = jnp.maximum(u_ref[...] * scale + shift, 0.0)
    h_ref[...] = hn
    for j in range(6):
        t_ref[j] = jnp.maximum(hn + eemb_ref[j, :], 0.0)


_bn_next_kernel = pl.pallas_call(
    _bn_next_body,
    grid=(NB,),
    in_specs=[
        pl.BlockSpec((RB, 128), lambda b: (b, 0)),
        pl.BlockSpec((NB, 8, 128), lambda b: (0, 0, 0)),
        pl.BlockSpec((1, 128), lambda b: (0, 0)),
        pl.BlockSpec((1, 128), lambda b: (0, 0)),
        pl.BlockSpec((8, 128), lambda b: (0, 0)),
    ],
    out_specs=[
        pl.BlockSpec((RB, 128), lambda b: (b, 0)),
        pl.BlockSpec((6, RB, 128), lambda b: (0, b, 0)),
    ],
    out_shape=[
        jax.ShapeDtypeStruct((N, 128), jnp.float32),
        jax.ShapeDtypeStruct((6, N, 128), jnp.float32),
    ],
)


def _bn_pool_body(u_ref, st_ref, g_ref, be_ref, batch_ref,
                  pooled_ref, cnt_ref):
    scale, shift = _bn_scale_shift(st_ref, g_ref, be_ref)
    hn = u_ref[...] * scale + shift
    b = pl.program_id(0)

    @pl.when(b == 0)
    def _():
        pooled_ref[...] = jnp.zeros((B, 128), jnp.float32)
        cnt_ref[...] = jnp.zeros((B, 128), jnp.float32)

    bb = batch_ref[0, 0, :]
    mask = (lax.broadcasted_iota(jnp.int32, (B, RB), 0)
            == bb[None, :]).astype(jnp.float32)
    pooled_ref[...] += jnp.dot(mask, hn, preferred_element_type=jnp.float32)
    cnt_ref[...] += jnp.broadcast_to(jnp.sum(mask, axis=1)[:, None], (B, 128))


_bn_pool_kernel = pl.pallas_call(
    _bn_pool_body,
    grid=(NB,),
    in_specs=[
        pl.BlockSpec((RB, 128), lambda b: (b, 0)),
        pl.BlockSpec((NB, 8, 128), lambda b: (0, 0, 0)),
        pl.BlockSpec((1, 128), lambda b: (0, 0)),
        pl.BlockSpec((1, 128), lambda b: (0, 0)),
        pl.BlockSpec((1, 1, RB), lambda b: (b, 0, 0)),
    ],
    out_specs=[
        pl.BlockSpec((B, 128), lambda b: (0, 0)),
        pl.BlockSpec((B, 128), lambda b: (0, 0)),
    ],
    out_shape=[
        jax.ShapeDtypeStruct((B, 128), jnp.float32),
        jax.ShapeDtypeStruct((B, 128), jnp.float32),
    ],
)


def _tail_body(pa_ref, ca_ref, pb_ref, cb_ref, ctx_ref,
               wc1_ref, bc1_ref, wc2_ref, bc2_ref,
               wo1_ref, bo1_ref, wo2_ref, bo2_ref,
               wm1_ref, bm1_ref, wm2_ref, bm2_ref, wm3_ref, bm3_ref,
               out_ref):
    def dense(x, w_ref, b_ref, act):
        y = (jnp.dot(x, w_ref[...], preferred_element_type=jnp.float32)
             + b_ref[0, :])
        return jnp.maximum(y, 0.0) if act else y

    pa = pa_ref[...] / jnp.maximum(ca_ref[...], 1.0)
    pb = pb_ref[...] / jnp.maximum(cb_ref[...], 1.0)
    ha = dense(dense(pa, wo1_ref, bo1_ref, True), wo2_ref, bo2_ref, False)
    hb = dense(dense(pb, wo1_ref, bo1_ref, True), wo2_ref, bo2_ref, False)
    ctx = dense(dense(ctx_ref[...], wc1_ref, bc1_ref, True),
                wc2_ref, bc2_ref, False)
    z = jnp.concatenate([ha, hb, ctx], axis=1)
    z = dense(z, wm1_ref, bm1_ref, True)
    z = dense(z, wm2_ref, bm2_ref, True)
    out_ref[...] = dense(z, wm3_ref, bm3_ref, False)


_tail_kernel = pl.pallas_call(
    _tail_body,
    out_shape=jax.ShapeDtypeStruct((B, 128), jnp.float32),
)


_sc_mesh = plsc.VectorSubcoreMesh(core_axis_name="c", subcore_axis_name="s")


def _sc_aggr_body(t_hbm, comb_hbm, dst_hbm, zeros_hbm, out_hbm,
                  idxv, dstv, buf0, buf1, shared,
                  sg0, sg1, ss0, ss1):
    c = lax.axis_index("c")
    s = lax.axis_index("s")
    wid = s * 2 + c
    bufs = (buf0, buf1)
    sgs = (sg0, sg1)
    sss = (ss0, ss1)

    def gather(j, b):
        return pltpu.async_copy(t_hbm.at[idxv.at[j]], bufs[b], sgs[b])

    def scatter(j, b):
        return pltpu.async_copy(bufs[b], shared.at[dstv.at[j]], sss[b],
                                add=True)

    def wait_gather(j, b):
        pltpu.make_async_copy(t_hbm.at[idxv.at[j]], bufs[b], sgs[b]).wait()

    def wait_scatter(j, b):
        pltpu.make_async_copy(bufs[b], shared.at[dstv.at[j]], sss[b]).wait()

    pltpu.sync_copy(zeros_hbm.at[pl.ds(s * RPT, RPT)],
                    shared.at[pl.ds(s * RPT, RPT)])
    pltpu.sync_copy(comb_hbm.at[wid], idxv)
    pltpu.sync_copy(dst_hbm.at[wid], dstv)
    plsc.subcore_barrier()

    for b in range(2):
        gather(jnp.int32(b), b)

    def body(k, carry):
        j = 2 * k
        wait_gather(j, 0)
        scatter(j, 0)
        wait_gather(j + 1, 1)
        scatter(j + 1, 1)
        wait_scatter(j, 0)
        gather(jnp.minimum(j + 2, NCHUNK - 1), 0)
        wait_scatter(j + 1, 1)
        gather(jnp.minimum(j + 3, NCHUNK - 1), 1)
        return carry

    lax.fori_loop(0, NCHUNK // 2, body, jnp.int32(0))
    for b in range(2):
        wait_gather(jnp.int32(NCHUNK - 1), b)
    plsc.subcore_barrier()
    pltpu.sync_copy(shared.at[pl.ds(s * RPT, RPT)],
                    out_hbm.at[c, pl.ds(s * RPT, RPT)])


_sc_aggr = functools.partial(
    pl.kernel,
    out_type=jax.ShapeDtypeStruct((2, NODE_PAD, 128), jnp.float32),
    mesh=_sc_mesh,
    scratch_types=[
        pltpu.VMEM((NCHUNK, CHUNK), jnp.int32),
        pltpu.VMEM((NCHUNK, CHUNK), jnp.int32),
        pltpu.VMEM((CHUNK, 128), jnp.float32),
        pltpu.VMEM((CHUNK, 128), jnp.float32),
        pltpu.VMEM_SHARED((NODE_PAD, 128), jnp.float32),
        pltpu.SemaphoreType.DMA,
        pltpu.SemaphoreType.DMA,
        pltpu.SemaphoreType.DMA,
        pltpu.SemaphoreType.DMA,
    ],
)(_sc_aggr_body)


def _pad_edges(v, fill):
    return jnp.concatenate(
        [v.astype(jnp.int32), jnp.full((E_PAD - E,), fill, jnp.int32)])


def kernel(xA, edge_indexA, edge_attrA, batchA, xB, edge_indexB, edge_attrB,
           batchB, context, params):
    x_emb = params['x_emb']
    e_emb8 = jnp.concatenate(
        [params['e_emb'], jnp.zeros((2, 128), jnp.float32)], axis=0)
    zeros = jnp.zeros((NODE_PAD, 128), jnp.float32)

    def branch(x, ei, ea, batch):
        src = _pad_edges(ei[0], 0).reshape(E_PAD // 128, 128)
        eav = _pad_edges(ea[:, 0], 0).reshape(E_PAD // 128, 128)
        dst3 = _pad_edges(ei[1], N).reshape(NW, NCHUNK, CHUNK)
        comb3 = _idx_kernel(src, eav).reshape(NW, NCHUNK, CHUNK)
        x3 = x[:, 0].astype(jnp.int32).reshape(NB, 1, RB)
        batch3 = batch.astype(jnp.int32).reshape(NB, 1, RB)

        h, t = _embed_kernel(x3, x_emb, e_emb8)
        for l in range(NUM_LAYER):
            p = params['gnn'][l]
            aggr = _sc_aggr(t.reshape(6 * N, 128), comb3, dst3, zeros)
            u, st = _mlp_kernel(h, aggr[0], aggr[1],
                                p['W1'], p['b1'].reshape(1, 256),
                                p['W2'], p['b2'].reshape(1, 128))
            g = p['gamma'].reshape(1, 128)
            be = p['beta'].reshape(1, 128)
            if l < NUM_LAYER - 1:
                h, t = _bn_next_kernel(u, st, g, be, e_emb8)
            else:
                pooled, cnt = _bn_pool_kernel(u, st, g, be, batch3)
        return pooled, cnt

    pooledA, cntA = branch(xA, edge_indexA, edge_attrA, batchA)
    pooledB, cntB = branch(xB, edge_indexB, edge_attrB, batchB)

    ctx_pad = jnp.pad(context, ((0, 0), (0, 1024 - context.shape[1])))
    wc1_pad = jnp.pad(params['Wc1'], ((0, 1024 - params['Wc1'].shape[0]),
                                      (0, 0)))
    wm3_pad = jnp.pad(params['Wm3'], ((0, 0), (0, 127)))
    bm3_pad = jnp.pad(params['bm3'], ((0, 127),))

    out = _tail_kernel(
        pooledA, cntA, pooledB, cntB, ctx_pad,
        wc1_pad, params['bc1'].reshape(1, 512),
        params['Wc2'], params['bc2'].reshape(1, 128),
        params['Wo1'], params['bo1'].reshape(1, 512),
        params['Wo2'], params['bo2'].reshape(1, 256),
        params['Wm1'], params['bm1'].reshape(1, 256),
        params['Wm2'], params['bm2'].reshape(1, 64),
        wm3_pad, bm3_pad.reshape(1, 128))
    return out[:, :1]


# speedup vs baseline: 2.3322x; 1.0013x over previous
import functools

import jax
import jax.numpy as jnp
from jax import lax
from jax.experimental import pallas as pl
from jax.experimental.pallas import tpu as pltpu
from jax.experimental.pallas import tpu_sc as plsc

N = 10000
E = 160000
B = 64
EMB = 128
NUM_LAYER = 5

NW = 32
CHUNK = 128
EPT = 5120
E_PAD = NW * EPT
NCHUNK = EPT // CHUNK
NODE_PAD = 10112
RPT = NODE_PAD // 16
RB = 400
NB = N // RB


def _idx_body(src_ref, ea_ref, out_ref):
    out_ref[...] = ea_ref[...] * N + src_ref[...]


_idx_kernel = pl.pallas_call(
    _idx_body,
    out_shape=jax.ShapeDtypeStruct((E_PAD // 128, 128), jnp.int32),
)


def _embed_body(x_ref, xemb_ref, eemb_ref, h_ref, t_ref):
    xb = x_ref[0, 0, :]
    oh = (xb[:, None] == lax.broadcasted_iota(jnp.int32, (RB, 120), 1))
    h = jnp.dot(oh.astype(jnp.float32), xemb_ref[...],
                preferred_element_type=jnp.float32)
    h_ref[...] = h
    for j in range(6):
        t_ref[j] = jnp.maximum(h + eemb_ref[j, :], 0.0)


_embed_kernel = pl.pallas_call(
    _embed_body,
    grid=(NB,),
    in_specs=[
        pl.BlockSpec((1, 1, RB), lambda b: (b, 0, 0)),
        pl.BlockSpec((120, 128), lambda b: (0, 0)),
        pl.BlockSpec((8, 128), lambda b: (0, 0)),
    ],
    out_specs=[
        pl.BlockSpec((RB, 128), lambda b: (b, 0)),
        pl.BlockSpec((6, RB, 128), lambda b: (0, b, 0)),
    ],
    out_shape=[
        jax.ShapeDtypeStruct((N, 128), jnp.float32),
        jax.ShapeDtypeStruct((6, N, 128), jnp.float32),
    ],
)


def _mlp_body(h_ref, a0_ref, a1_ref, w1_ref, b1_ref, w2_ref, b2_ref,
              u_ref, st_ref):
    z = h_ref[...] + a0_ref[...] + a1_ref[...]
    t = jnp.maximum(
        jnp.dot(z, w1_ref[...], preferred_element_type=jnp.float32)
        + b1_ref[0, :], 0.0)
    u = (jnp.dot(t, w2_ref[...], preferred_element_type=jnp.float32)
         + b2_ref[0, :])
    u_ref[...] = u
    s1 = jnp.sum(u, axis=0, keepdims=True)
    s2 = jnp.sum(u * u, axis=0, keepdims=True)
    st_ref[0] = jnp.concatenate([s1, s2, jnp.zeros((6, 128), jnp.float32)], 0)


_mlp_kernel = pl.pallas_call(
    _mlp_body,
    grid=(NB,),
    in_specs=[
        pl.BlockSpec((RB, 128), lambda b: (b, 0)),
        pl.BlockSpec((RB, 128), lambda b: (b, 0)),
        pl.BlockSpec((RB, 128), lambda b: (b, 0)),
        pl.BlockSpec((128, 256), lambda b: (0, 0)),
        pl.BlockSpec((1, 256), lambda b: (0, 0)),
        pl.BlockSpec((256, 128), lambda b: (0, 0)),
        pl.BlockSpec((1, 128), lambda b: (0, 0)),
    ],
    out_specs=[
        pl.BlockSpec((RB, 128), lambda b: (b, 0)),
        pl.BlockSpec((1, 8, 128), lambda b: (b, 0, 0)),
    ],
    out_shape=[
        jax.ShapeDtypeStruct((N, 128), jnp.float32),
        jax.ShapeDtypeStruct((NB, 8, 128), jnp.float32),
    ],
)


def _bn_scale_shift(st_ref, g_ref, be_ref):
    s = st_ref[...]
    mu = jnp.sum(s[:, 0, :], axis=0) / N
    ex2 = jnp.sum(s[:, 1, :], axis=0) / N
    var = ex2 - mu * mu
    scale = g_ref[0, :] * lax.rsqrt(var + 1e-5)
    shift = be_ref[0, :] - mu * scale
    return scale, shift


def _bn_next_body(u_ref, st_ref, g_ref, be_ref, eemb_ref, h_ref, t_ref):
    scale, shift = _bn_scale_shift(st_ref, g_ref, be_ref)
    hn = jnp.maximum(u_ref[...] * scale + shift, 0.0)
    h_ref[...] = hn
    for j in range(6):
        t_ref[j] = jnp.maximum(hn + eemb_ref[j, :], 0.0)


_bn_next_kernel = pl.pallas_call(
    _bn_next_body,
    grid=(NB,),
    in_specs=[
        pl.BlockSpec((RB, 128), lambda b: (b, 0)),
        pl.BlockSpec((NB, 8, 128), lambda b: (0, 0, 0)),
        pl.BlockSpec((1, 128), lambda b: (0, 0)),
        pl.BlockSpec((1, 128), lambda b: (0, 0)),
        pl.BlockSpec((8, 128), lambda b: (0, 0)),
    ],
    out_specs=[
        pl.BlockSpec((RB, 128), lambda b: (b, 0)),
        pl.BlockSpec((6, RB, 128), lambda b: (0, b, 0)),
    ],
    out_shape=[
        jax.ShapeDtypeStruct((N, 128), jnp.float32),
        jax.ShapeDtypeStruct((6, N, 128), jnp.float32),
    ],
)


def _bn_pool_body(u_ref, st_ref, g_ref, be_ref, batch_ref,
                  pooled_ref, cnt_ref):
    scale, shift = _bn_scale_shift(st_ref, g_ref, be_ref)
    hn = u_ref[...] * scale + shift
    b = pl.program_id(0)

    @pl.when(b == 0)
    def _():
        pooled_ref[...] = jnp.zeros((B, 128), jnp.float32)
        cnt_ref[...] = jnp.zeros((B, 128), jnp.float32)

    bb = batch_ref[0, 0, :]
    mask = (lax.broadcasted_iota(jnp.int32, (B, RB), 0)
            == bb[None, :]).astype(jnp.float32)
    pooled_ref[...] += jnp.dot(mask, hn, preferred_element_type=jnp.float32)
    cnt_ref[...] += jnp.broadcast_to(jnp.sum(mask, axis=1)[:, None], (B, 128))


_bn_pool_kernel = pl.pallas_call(
    _bn_pool_body,
    grid=(NB,),
    in_specs=[
        pl.BlockSpec((RB, 128), lambda b: (b, 0)),
        pl.BlockSpec((NB, 8, 128), lambda b: (0, 0, 0)),
        pl.BlockSpec((1, 128), lambda b: (0, 0)),
        pl.BlockSpec((1, 128), lambda b: (0, 0)),
        pl.BlockSpec((1, 1, RB), lambda b: (b, 0, 0)),
    ],
    out_specs=[
        pl.BlockSpec((B, 128), lambda b: (0, 0)),
        pl.BlockSpec((B, 128), lambda b: (0, 0)),
    ],
    out_shape=[
        jax.ShapeDtypeStruct((B, 128), jnp.float32),
        jax.ShapeDtypeStruct((B, 128), jnp.float32),
    ],
)


def _tail_body(pa_ref, ca_ref, pb_ref, cb_ref, ctx_ref,
               wc1_ref, bc1_ref, wc2_ref, bc2_ref,
               wo1_ref, bo1_ref, wo2_ref, bo2_ref,
               wm1_ref, bm1_ref, wm2_ref, bm2_ref, wm3_ref, bm3_ref,
               out_ref):
    def dense(x, w_ref, b_ref, act):
        y = (jnp.dot(x, w_ref[...], preferred_element_type=jnp.float32)
             + b_ref[0, :])
        return jnp.maximum(y, 0.0) if act else y

    pa = pa_ref[...] / jnp.maximum(ca_ref[...], 1.0)
    pb = pb_ref[...] / jnp.maximum(cb_ref[...], 1.0)
    ha = dense(dense(pa, wo1_ref, bo1_ref, True), wo2_ref, bo2_ref, False)
    hb = dense(dense(pb, wo1_ref, bo1_ref, True), wo2_ref, bo2_ref, False)
    ctx = dense(dense(ctx_ref[...], wc1_ref, bc1_ref, True),
                wc2_ref, bc2_ref, False)
    z = jnp.concatenate([ha, hb, ctx], axis=1)
    z = dense(z, wm1_ref, bm1_ref, True)
    z = dense(z, wm2_ref, bm2_ref, True)
    out_ref[...] = dense(z, wm3_ref, bm3_ref, False)


_tail_kernel = pl.pallas_call(
    _tail_body,
    out_shape=jax.ShapeDtypeStruct((B, 128), jnp.float32),
)


_sc_mesh = plsc.VectorSubcoreMesh(core_axis_name="c", subcore_axis_name="s")


def _sc_aggr_body(t_hbm, comb_hbm, dst_hbm, zeros_hbm, out_hbm,
                  idxv, dstv, buf0, buf1, shared,
                  sg0, sg1, ss0, ss1):
    c = lax.axis_index("c")
    s = lax.axis_index("s")
    wid = s * 2 + c
    bufs = (buf0, buf1)
    sgs = (sg0, sg1)
    sss = (ss0, ss1)

    def gather(j, b):
        return pltpu.async_copy(t_hbm.at[idxv.at[j]], bufs[b], sgs[b])

    def scatter(j, b):
        return pltpu.async_copy(bufs[b], shared.at[dstv.at[j]], sss[b],
                                add=True)

    def wait_gather(j, b):
        pltpu.make_async_copy(t_hbm.at[idxv.at[j]], bufs[b], sgs[b]).wait()

    def wait_scatter(j, b):
        pltpu.make_async_copy(bufs[b], shared.at[dstv.at[j]], sss[b]).wait()

    pltpu.sync_copy(zeros_hbm.at[pl.ds(s * RPT, RPT)],
                    shared.at[pl.ds(s * RPT, RPT)])
    pltpu.sync_copy(comb_hbm.at[wid], idxv)
    pltpu.sync_copy(dst_hbm.at[wid], dstv)
    plsc.subcore_barrier()

    for b in range(2):
        gather(jnp.int32(b), b)

    def body(k, carry):
        j = 2 * k
        wait_gather(j, 0)
        scatter(j, 0)
        wait_gather(j + 1, 1)
        scatter(j + 1, 1)
        wait_scatter(j, 0)
        gather(jnp.minimum(j + 2, NCHUNK - 1), 0)
        wait_scatter(j + 1, 1)
        gather(jnp.minimum(j + 3, NCHUNK - 1), 1)
        return carry

    lax.fori_loop(0, NCHUNK // 2, body, jnp.int32(0))
    for b in range(2):
        wait_gather(jnp.int32(NCHUNK - 1), b)
    plsc.subcore_barrier()
    pltpu.sync_copy(shared.at[pl.ds(s * RPT, RPT)],
                    out_hbm.at[c, pl.ds(s * RPT, RPT)])


_sc_aggr = functools.partial(
    pl.kernel,
    out_type=jax.ShapeDtypeStruct((2, NODE_PAD, 128), jnp.float32),
    mesh=_sc_mesh,
    scratch_types=[
        pltpu.VMEM((NCHUNK, CHUNK), jnp.int32),
        pltpu.VMEM((NCHUNK, CHUNK), jnp.int32),
        pltpu.VMEM((CHUNK, 128), jnp.float32),
        pltpu.VMEM((CHUNK, 128), jnp.float32),
        pltpu.VMEM_SHARED((NODE_PAD, 128), jnp.float32),
        pltpu.SemaphoreType.DMA,
        pltpu.SemaphoreType.DMA,
        pltpu.SemaphoreType.DMA,
        pltpu.SemaphoreType.DMA,
    ],
)(_sc_aggr_body)


def _pad_edges(v, fill):
    return jnp.concatenate(
        [v.astype(jnp.int32), jnp.full((E_PAD - E,), fill, jnp.int32)])


def kernel(xA, edge_indexA, edge_attrA, batchA, xB, edge_indexB, edge_attrB,
           batchB, context, params):
    x_emb = params['x_emb']
    e_emb8 = jnp.concatenate(
        [params['e_emb'], jnp.zeros((2, 128), jnp.float32)], axis=0)
    zeros = jnp.zeros((NODE_PAD, 128), jnp.float32)

    def prep(x, ei, ea, batch):
        src = _pad_edges(ei[0], 0).reshape(E_PAD // 128, 128)
        eav = _pad_edges(ea[:, 0], 0).reshape(E_PAD // 128, 128)
        dst3 = _pad_edges(ei[1], N).reshape(NW, NCHUNK, CHUNK)
        comb3 = _idx_kernel(src, eav).reshape(NW, NCHUNK, CHUNK)
        x3 = x[:, 0].astype(jnp.int32).reshape(NB, 1, RB)
        batch3 = batch.astype(jnp.int32).reshape(NB, 1, RB)
        h, t = _embed_kernel(x3, x_emb, e_emb8)
        return {'comb': comb3, 'dst': dst3, 'batch': batch3, 'h': h, 't': t}

    sA = prep(xA, edge_indexA, edge_attrA, batchA)
    sB = prep(xB, edge_indexB, edge_attrB, batchB)
    for l in range(NUM_LAYER):
        p = params['gnn'][l]
        b1 = p['b1'].reshape(1, 256)
        b2 = p['b2'].reshape(1, 128)
        g = p['gamma'].reshape(1, 128)
        be = p['beta'].reshape(1, 128)
        for st_ in (sA, sB):
            st_['aggr'] = _sc_aggr(st_['t'].reshape(6 * N, 128),
                                   st_['comb'], st_['dst'], zeros)
        for st_ in (sA, sB):
            aggr = st_['aggr']
            u, st = _mlp_kernel(st_['h'], aggr[0], aggr[1],
                                p['W1'], b1, p['W2'], b2)
            if l < NUM_LAYER - 1:
                st_['h'], st_['t'] = _bn_next_kernel(u, st, g, be, e_emb8)
            else:
                st_['pooled'], st_['cnt'] = _bn_pool_kernel(
                    u, st, g, be, st_['batch'])
    pooledA, cntA = sA['pooled'], sA['cnt']
    pooledB, cntB = sB['pooled'], sB['cnt']

    ctx_pad = jnp.pad(context, ((0, 0), (0, 1024 - context.shape[1])))
    wc1_pad = jnp.pad(params['Wc1'], ((0, 1024 - params['Wc1'].shape[0]),
                                      (0, 0)))
    wm3_pad = jnp.pad(params['Wm3'], ((0, 0), (0, 127)))
    bm3_pad = jnp.pad(params['bm3'], ((0, 127),))

    out = _tail_kernel(
        pooledA, cntA, pooledB, cntB, ctx_pad,
        wc1_pad, params['bc1'].reshape(1, 512),
        params['Wc2'], params['bc2'].reshape(1, 128),
        params['Wo1'], params['bo1'].reshape(1, 512),
        params['Wo2'], params['bo2'].reshape(1, 256),
        params['Wm1'], params['bm1'].reshape(1, 256),
        params['Wm2'], params['bm2'].reshape(1, 64),
        wm3_pad, bm3_pad.reshape(1, 128))
    return out[:, :1]


# speedup vs baseline: 2.7218x; 1.1670x over previous
import functools

import jax
import jax.numpy as jnp
from jax import lax
from jax.experimental import pallas as pl
from jax.experimental.pallas import tpu as pltpu
from jax.experimental.pallas import tpu_sc as plsc

N = 10000
E = 160000
B = 64
EMB = 128
NUM_LAYER = 5

NW = 32
CHUNK = 128
EPT = 5120
E_PAD = NW * EPT
NCHUNK = EPT // CHUNK
NODE_PAD = 10112
RPT = NODE_PAD // 16
RB = 1000
NB = N // RB


def _idx_body(src_ref, ea_ref, out_ref):
    out_ref[...] = ea_ref[...] * N + src_ref[...]


_idx_kernel = pl.pallas_call(
    _idx_body,
    out_shape=jax.ShapeDtypeStruct((E_PAD // 128, 128), jnp.int32),
)


def _embed_body(x_ref, xemb_ref, eemb_ref, h_ref, t_ref):
    xb = x_ref[0, 0, :]
    oh = (xb[:, None] == lax.broadcasted_iota(jnp.int32, (RB, 120), 1))
    h = jnp.dot(oh.astype(jnp.float32), xemb_ref[...],
                preferred_element_type=jnp.float32)
    h_ref[...] = h
    for j in range(6):
        t_ref[j] = jnp.maximum(h + eemb_ref[j, :], 0.0)


_embed_kernel = pl.pallas_call(
    _embed_body,
    grid=(NB,),
    in_specs=[
        pl.BlockSpec((1, 1, RB), lambda b: (b, 0, 0)),
        pl.BlockSpec((120, 128), lambda b: (0, 0)),
        pl.BlockSpec((8, 128), lambda b: (0, 0)),
    ],
    out_specs=[
        pl.BlockSpec((RB, 128), lambda b: (b, 0)),
        pl.BlockSpec((6, RB, 128), lambda b: (0, b, 0)),
    ],
    out_shape=[
        jax.ShapeDtypeStruct((N, 128), jnp.float32),
        jax.ShapeDtypeStruct((6, N, 128), jnp.float32),
    ],
)


def _mlp_body(h_ref, a0_ref, a1_ref, w1_ref, b1_ref, w2_ref, b2_ref,
              u_ref, st_ref):
    z = h_ref[...] + a0_ref[...] + a1_ref[...]
    t = jnp.maximum(
        jnp.dot(z, w1_ref[...], preferred_element_type=jnp.float32)
        + b1_ref[0, :], 0.0)
    u = (jnp.dot(t, w2_ref[...], preferred_element_type=jnp.float32)
         + b2_ref[0, :])
    u_ref[...] = u
    s1 = jnp.sum(u, axis=0, keepdims=True)
    s2 = jnp.sum(u * u, axis=0, keepdims=True)
    st_ref[0] = jnp.concatenate([s1, s2, jnp.zeros((6, 128), jnp.float32)], 0)


_mlp_kernel = pl.pallas_call(
    _mlp_body,
    grid=(NB,),
    in_specs=[
        pl.BlockSpec((RB, 128), lambda b: (b, 0)),
        pl.BlockSpec((RB, 128), lambda b: (b, 0)),
        pl.BlockSpec((RB, 128), lambda b: (b, 0)),
        pl.BlockSpec((128, 256), lambda b: (0, 0)),
        pl.BlockSpec((1, 256), lambda b: (0, 0)),
        pl.BlockSpec((256, 128), lambda b: (0, 0)),
        pl.BlockSpec((1, 128), lambda b: (0, 0)),
    ],
    out_specs=[
        pl.BlockSpec((RB, 128), lambda b: (b, 0)),
        pl.BlockSpec((1, 8, 128), lambda b: (b, 0, 0)),
    ],
    out_shape=[
        jax.ShapeDtypeStruct((N, 128), jnp.float32),
        jax.ShapeDtypeStruct((NB, 8, 128), jnp.float32),
    ],
)


def _bn_scale_shift(st_ref, g_ref, be_ref):
    s = st_ref[...]
    mu = jnp.sum(s[:, 0, :], axis=0) / N
    ex2 = jnp.sum(s[:, 1, :], axis=0) / N
    var = ex2 - mu * mu
    scale = g_ref[0, :] * lax.rsqrt(var + 1e-5)
    shift = be_ref[0, :] - mu * scale
    return scale, shift


def _bn_next_body(u_ref, st_ref, g_ref, be_ref, eemb_ref, h_ref, t_ref):
    scale, shift = _bn_scale_shift(st_ref, g_ref, be_ref)
    hn = jnp.maximum(u_ref[...] * scale + shift, 0.0)
    h_ref[...] = hn
    for j in range(6):
        t_ref[j] = jnp.maximum(hn + eemb_ref[j, :], 0.0)


_bn_next_kernel = pl.pallas_call(
    _bn_next_body,
    grid=(NB,),
    in_specs=[
        pl.BlockSpec((RB, 128), lambda b: (b, 0)),
        pl.BlockSpec((NB, 8, 128), lambda b: (0, 0, 0)),
        pl.BlockSpec((1, 128), lambda b: (0, 0)),
        pl.BlockSpec((1, 128), lambda b: (0, 0)),
        pl.BlockSpec((8, 128), lambda b: (0, 0)),
    ],
    out_specs=[
        pl.BlockSpec((RB, 128), lambda b: (b, 0)),
        pl.BlockSpec((6, RB, 128), lambda b: (0, b, 0)),
    ],
    out_shape=[
        jax.ShapeDtypeStruct((N, 128), jnp.float32),
        jax.ShapeDtypeStruct((6, N, 128), jnp.float32),
    ],
)


def _bn_pool_body(u_ref, st_ref, g_ref, be_ref, batch_ref,
                  pooled_ref, cnt_ref):
    scale, shift = _bn_scale_shift(st_ref, g_ref, be_ref)
    hn = u_ref[...] * scale + shift
    b = pl.program_id(0)

    @pl.when(b == 0)
    def _():
        pooled_ref[...] = jnp.zeros((B, 128), jnp.float32)
        cnt_ref[...] = jnp.zeros((B, 128), jnp.float32)

    bb = batch_ref[0, 0, :]
    mask = (lax.broadcasted_iota(jnp.int32, (B, RB), 0)
            == bb[None, :]).astype(jnp.float32)
    pooled_ref[...] += jnp.dot(mask, hn, preferred_element_type=jnp.float32)
    cnt_ref[...] += jnp.broadcast_to(jnp.sum(mask, axis=1)[:, None], (B, 128))


_bn_pool_kernel = pl.pallas_call(
    _bn_pool_body,
    grid=(NB,),
    in_specs=[
        pl.BlockSpec((RB, 128), lambda b: (b, 0)),
        pl.BlockSpec((NB, 8, 128), lambda b: (0, 0, 0)),
        pl.BlockSpec((1, 128), lambda b: (0, 0)),
        pl.BlockSpec((1, 128), lambda b: (0, 0)),
        pl.BlockSpec((1, 1, RB), lambda b: (b, 0, 0)),
    ],
    out_specs=[
        pl.BlockSpec((B, 128), lambda b: (0, 0)),
        pl.BlockSpec((B, 128), lambda b: (0, 0)),
    ],
    out_shape=[
        jax.ShapeDtypeStruct((B, 128), jnp.float32),
        jax.ShapeDtypeStruct((B, 128), jnp.float32),
    ],
)


def _tail_body(pa_ref, ca_ref, pb_ref, cb_ref, ctx_ref,
               wc1_ref, bc1_ref, wc2_ref, bc2_ref,
               wo1_ref, bo1_ref, wo2_ref, bo2_ref,
               wm1_ref, bm1_ref, wm2_ref, bm2_ref, wm3_ref, bm3_ref,
               out_ref):
    def dense(x, w_ref, b_ref, act):
        y = (jnp.dot(x, w_ref[...], preferred_element_type=jnp.float32)
             + b_ref[0, :])
        return jnp.maximum(y, 0.0) if act else y

    pa = pa_ref[...] / jnp.maximum(ca_ref[...], 1.0)
    pb = pb_ref[...] / jnp.maximum(cb_ref[...], 1.0)
    ha = dense(dense(pa, wo1_ref, bo1_ref, True), wo2_ref, bo2_ref, False)
    hb = dense(dense(pb, wo1_ref, bo1_ref, True), wo2_ref, bo2_ref, False)
    ctx = dense(dense(ctx_ref[...], wc1_ref, bc1_ref, True),
                wc2_ref, bc2_ref, False)
    z = jnp.concatenate([ha, hb, ctx], axis=1)
    z = dense(z, wm1_ref, bm1_ref, True)
    z = dense(z, wm2_ref, bm2_ref, True)
    out_ref[...] = dense(z, wm3_ref, bm3_ref, False)


_tail_kernel = pl.pallas_call(
    _tail_body,
    out_shape=jax.ShapeDtypeStruct((B, 128), jnp.float32),
)


_sc_mesh = plsc.VectorSubcoreMesh(core_axis_name="c", subcore_axis_name="s")


def _sc_aggr_body(t_hbm, comb_hbm, dst_hbm, zeros_hbm, out_hbm,
                  idxv, dstv, buf0, buf1, shared,
                  sg0, sg1, ss0, ss1):
    c = lax.axis_index("c")
    s = lax.axis_index("s")
    wid = s * 2 + c
    bufs = (buf0, buf1)
    sgs = (sg0, sg1)
    sss = (ss0, ss1)

    def gather(j, b):
        return pltpu.async_copy(t_hbm.at[idxv.at[j]], bufs[b], sgs[b])

    def scatter(j, b):
        return pltpu.async_copy(bufs[b], shared.at[dstv.at[j]], sss[b],
                                add=True)

    def wait_gather(j, b):
        pltpu.make_async_copy(t_hbm.at[idxv.at[j]], bufs[b], sgs[b]).wait()

    def wait_scatter(j, b):
        pltpu.make_async_copy(bufs[b], shared.at[dstv.at[j]], sss[b]).wait()

    pltpu.sync_copy(zeros_hbm.at[pl.ds(s * RPT, RPT)],
                    shared.at[pl.ds(s * RPT, RPT)])
    pltpu.sync_copy(comb_hbm.at[wid], idxv)
    pltpu.sync_copy(dst_hbm.at[wid], dstv)
    plsc.subcore_barrier()

    for b in range(2):
        gather(jnp.int32(b), b)

    def body(k, carry):
        j = 2 * k
        wait_gather(j, 0)
        scatter(j, 0)
        wait_gather(j + 1, 1)
        scatter(j + 1, 1)
        wait_scatter(j, 0)
        gather(jnp.minimum(j + 2, NCHUNK - 1), 0)
        wait_scatter(j + 1, 1)
        gather(jnp.minimum(j + 3, NCHUNK - 1), 1)
        return carry

    lax.fori_loop(0, NCHUNK // 2, body, jnp.int32(0))
    for b in range(2):
        wait_gather(jnp.int32(NCHUNK - 1), b)
    plsc.subcore_barrier()
    pltpu.sync_copy(shared.at[pl.ds(s * RPT, RPT)],
                    out_hbm.at[c, pl.ds(s * RPT, RPT)])


_sc_aggr = functools.partial(
    pl.kernel,
    out_type=jax.ShapeDtypeStruct((2, NODE_PAD, 128), jnp.float32),
    mesh=_sc_mesh,
    scratch_types=[
        pltpu.VMEM((NCHUNK, CHUNK), jnp.int32),
        pltpu.VMEM((NCHUNK, CHUNK), jnp.int32),
        pltpu.VMEM((CHUNK, 128), jnp.float32),
        pltpu.VMEM((CHUNK, 128), jnp.float32),
        pltpu.VMEM_SHARED((NODE_PAD, 128), jnp.float32),
        pltpu.SemaphoreType.DMA,
        pltpu.SemaphoreType.DMA,
        pltpu.SemaphoreType.DMA,
        pltpu.SemaphoreType.DMA,
    ],
)(_sc_aggr_body)


def _pad_edges(v, fill):
    return jnp.concatenate(
        [v.astype(jnp.int32), jnp.full((E_PAD - E,), fill, jnp.int32)])


def kernel(xA, edge_indexA, edge_attrA, batchA, xB, edge_indexB, edge_attrB,
           batchB, context, params):
    x_emb = params['x_emb']
    e_emb8 = jnp.concatenate(
        [params['e_emb'], jnp.zeros((2, 128), jnp.float32)], axis=0)
    zeros = jnp.zeros((NODE_PAD, 128), jnp.float32)

    def prep(x, ei, ea, batch):
        src = _pad_edges(ei[0], 0).reshape(E_PAD // 128, 128)
        eav = _pad_edges(ea[:, 0], 0).reshape(E_PAD // 128, 128)
        dst3 = _pad_edges(ei[1], N).reshape(NW, NCHUNK, CHUNK)
        comb3 = _idx_kernel(src, eav).reshape(NW, NCHUNK, CHUNK)
        x3 = x[:, 0].astype(jnp.int32).reshape(NB, 1, RB)
        batch3 = batch.astype(jnp.int32).reshape(NB, 1, RB)
        h, t = _embed_kernel(x3, x_emb, e_emb8)
        return {'comb': comb3, 'dst': dst3, 'batch': batch3, 'h': h, 't': t}

    sA = prep(xA, edge_indexA, edge_attrA, batchA)
    sB = prep(xB, edge_indexB, edge_attrB, batchB)
    for l in range(NUM_LAYER):
        p = params['gnn'][l]
        b1 = p['b1'].reshape(1, 256)
        b2 = p['b2'].reshape(1, 128)
        g = p['gamma'].reshape(1, 128)
        be = p['beta'].reshape(1, 128)
        for st_ in (sA, sB):
            st_['aggr'] = _sc_aggr(st_['t'].reshape(6 * N, 128),
                                   st_['comb'], st_['dst'], zeros)
        for st_ in (sA, sB):
            aggr = st_['aggr']
            u, st = _mlp_kernel(st_['h'], aggr[0], aggr[1],
                                p['W1'], b1, p['W2'], b2)
            if l < NUM_LAYER - 1:
                st_['h'], st_['t'] = _bn_next_kernel(u, st, g, be, e_emb8)
            else:
                st_['pooled'], st_['cnt'] = _bn_pool_kernel(
                    u, st, g, be, st_['batch'])
    pooledA, cntA = sA['pooled'], sA['cnt']
    pooledB, cntB = sB['pooled'], sB['cnt']

    ctx_pad = jnp.pad(context, ((0, 0), (0, 1024 - context.shape[1])))
    wc1_pad = jnp.pad(params['Wc1'], ((0, 1024 - params['Wc1'].shape[0]),
                                      (0, 0)))
    wm3_pad = jnp.pad(params['Wm3'], ((0, 0), (0, 127)))
    bm3_pad = jnp.pad(params['bm3'], ((0, 127),))

    out = _tail_kernel(
        pooledA, cntA, pooledB, cntB, ctx_pad,
        wc1_pad, params['bc1'].reshape(1, 512),
        params['Wc2'], params['bc2'].reshape(1, 128),
        params['Wo1'], params['bo1'].reshape(1, 512),
        params['Wo2'], params['bo2'].reshape(1, 256),
        params['Wm1'], params['bm1'].reshape(1, 256),
        params['Wm2'], params['bm2'].reshape(1, 64),
        wm3_pad, bm3_pad.reshape(1, 128))
    return out[:, :1]


# speedup vs baseline: 2.7332x; 1.0042x over previous
import functools

import jax
import jax.numpy as jnp
from jax import lax
from jax.experimental import pallas as pl
from jax.experimental.pallas import tpu as pltpu
from jax.experimental.pallas import tpu_sc as plsc

N = 10000
E = 160000
B = 64
EMB = 128
NUM_LAYER = 5

NW = 32
CHUNK = 128
EPT = 5120
E_PAD = NW * EPT
NCHUNK = EPT // CHUNK
NODE_PAD = 10112
RPT = NODE_PAD // 16
RB = 2000
NB = N // RB


def _idx_body(src_ref, ea_ref, out_ref):
    out_ref[...] = ea_ref[...] * N + src_ref[...]


_idx_kernel = pl.pallas_call(
    _idx_body,
    out_shape=jax.ShapeDtypeStruct((E_PAD // 128, 128), jnp.int32),
)


def _embed_body(x_ref, xemb_ref, eemb_ref, h_ref, t_ref):
    xb = x_ref[0, 0, :]
    oh = (xb[:, None] == lax.broadcasted_iota(jnp.int32, (RB, 120), 1))
    h = jnp.dot(oh.astype(jnp.float32), xemb_ref[...],
                preferred_element_type=jnp.float32)
    h_ref[...] = h
    for j in range(6):
        t_ref[j] = jnp.maximum(h + eemb_ref[j, :], 0.0)


_embed_kernel = pl.pallas_call(
    _embed_body,
    grid=(NB,),
    in_specs=[
        pl.BlockSpec((1, 1, RB), lambda b: (b, 0, 0)),
        pl.BlockSpec((120, 128), lambda b: (0, 0)),
        pl.BlockSpec((8, 128), lambda b: (0, 0)),
    ],
    out_specs=[
        pl.BlockSpec((RB, 128), lambda b: (b, 0)),
        pl.BlockSpec((6, RB, 128), lambda b: (0, b, 0)),
    ],
    out_shape=[
        jax.ShapeDtypeStruct((N, 128), jnp.float32),
        jax.ShapeDtypeStruct((6, N, 128), jnp.float32),
    ],
)


def _mlp_body(h_ref, a0_ref, a1_ref, w1_ref, b1_ref, w2_ref, b2_ref,
              u_ref, st_ref):
    z = h_ref[...] + a0_ref[...] + a1_ref[...]
    t = jnp.maximum(
        jnp.dot(z, w1_ref[...], preferred_element_type=jnp.float32)
        + b1_ref[0, :], 0.0)
    u = (jnp.dot(t, w2_ref[...], preferred_element_type=jnp.float32)
         + b2_ref[0, :])
    u_ref[...] = u
    s1 = jnp.sum(u, axis=0, keepdims=True)
    s2 = jnp.sum(u * u, axis=0, keepdims=True)
    st_ref[0] = jnp.concatenate([s1, s2, jnp.zeros((6, 128), jnp.float32)], 0)


_mlp_kernel = pl.pallas_call(
    _mlp_body,
    grid=(NB,),
    in_specs=[
        pl.BlockSpec((RB, 128), lambda b: (b, 0)),
        pl.BlockSpec((RB, 128), lambda b: (b, 0)),
        pl.BlockSpec((RB, 128), lambda b: (b, 0)),
        pl.BlockSpec((128, 256), lambda b: (0, 0)),
        pl.BlockSpec((1, 256), lambda b: (0, 0)),
        pl.BlockSpec((256, 128), lambda b: (0, 0)),
        pl.BlockSpec((1, 128), lambda b: (0, 0)),
    ],
    out_specs=[
        pl.BlockSpec((RB, 128), lambda b: (b, 0)),
        pl.BlockSpec((1, 8, 128), lambda b: (b, 0, 0)),
    ],
    out_shape=[
        jax.ShapeDtypeStruct((N, 128), jnp.float32),
        jax.ShapeDtypeStruct((NB, 8, 128), jnp.float32),
    ],
)


def _bn_scale_shift(st_ref, g_ref, be_ref):
    s = st_ref[...]
    mu = jnp.sum(s[:, 0, :], axis=0) / N
    ex2 = jnp.sum(s[:, 1, :], axis=0) / N
    var = ex2 - mu * mu
    scale = g_ref[0, :] * lax.rsqrt(var + 1e-5)
    shift = be_ref[0, :] - mu * scale
    return scale, shift


def _bn_next_body(u_ref, st_ref, g_ref, be_ref, eemb_ref, h_ref, t_ref):
    scale, shift = _bn_scale_shift(st_ref, g_ref, be_ref)
    hn = jnp.maximum(u_ref[...] * scale + shift, 0.0)
    h_ref[...] = hn
    for j in range(6):
        t_ref[j] = jnp.maximum(hn + eemb_ref[j, :], 0.0)


_bn_next_kernel = pl.pallas_call(
    _bn_next_body,
    grid=(NB,),
    in_specs=[
        pl.BlockSpec((RB, 128), lambda b: (b, 0)),
        pl.BlockSpec((NB, 8, 128), lambda b: (0, 0, 0)),
        pl.BlockSpec((1, 128), lambda b: (0, 0)),
        pl.BlockSpec((1, 128), lambda b: (0, 0)),
        pl.BlockSpec((8, 128), lambda b: (0, 0)),
    ],
    out_specs=[
        pl.BlockSpec((RB, 128), lambda b: (b, 0)),
        pl.BlockSpec((6, RB, 128), lambda b: (0, b, 0)),
    ],
    out_shape=[
        jax.ShapeDtypeStruct((N, 128), jnp.float32),
        jax.ShapeDtypeStruct((6, N, 128), jnp.float32),
    ],
)


def _bn_pool_body(u_ref, st_ref, g_ref, be_ref, batch_ref,
                  pooled_ref, cnt_ref):
    scale, shift = _bn_scale_shift(st_ref, g_ref, be_ref)
    hn = u_ref[...] * scale + shift
    b = pl.program_id(0)

    @pl.when(b == 0)
    def _():
        pooled_ref[...] = jnp.zeros((B, 128), jnp.float32)
        cnt_ref[...] = jnp.zeros((B, 128), jnp.float32)

    bb = batch_ref[0, 0, :]
    mask = (lax.broadcasted_iota(jnp.int32, (B, RB), 0)
            == bb[None, :]).astype(jnp.float32)
    pooled_ref[...] += jnp.dot(mask, hn, preferred_element_type=jnp.float32)
    cnt_ref[...] += jnp.broadcast_to(jnp.sum(mask, axis=1)[:, None], (B, 128))


_bn_pool_kernel = pl.pallas_call(
    _bn_pool_body,
    grid=(NB,),
    in_specs=[
        pl.BlockSpec((RB, 128), lambda b: (b, 0)),
        pl.BlockSpec((NB, 8, 128), lambda b: (0, 0, 0)),
        pl.BlockSpec((1, 128), lambda b: (0, 0)),
        pl.BlockSpec((1, 128), lambda b: (0, 0)),
        pl.BlockSpec((1, 1, RB), lambda b: (b, 0, 0)),
    ],
    out_specs=[
        pl.BlockSpec((B, 128), lambda b: (0, 0)),
        pl.BlockSpec((B, 128), lambda b: (0, 0)),
    ],
    out_shape=[
        jax.ShapeDtypeStruct((B, 128), jnp.float32),
        jax.ShapeDtypeStruct((B, 128), jnp.float32),
    ],
)


def _tail_body(pa_ref, ca_ref, pb_ref, cb_ref, ctx_ref,
               wc1_ref, bc1_ref, wc2_ref, bc2_ref,
               wo1_ref, bo1_ref, wo2_ref, bo2_ref,
               wm1_ref, bm1_ref, wm2_ref, bm2_ref, wm3_ref, bm3_ref,
               out_ref):
    def dense(x, w_ref, b_ref, act):
        y = (jnp.dot(x, w_ref[...], preferred_element_type=jnp.float32)
             + b_ref[0, :])
        return jnp.maximum(y, 0.0) if act else y

    pa = pa_ref[...] / jnp.maximum(ca_ref[...], 1.0)
    pb = pb_ref[...] / jnp.maximum(cb_ref[...], 1.0)
    ha = dense(dense(pa, wo1_ref, bo1_ref, True), wo2_ref, bo2_ref, False)
    hb = dense(dense(pb, wo1_ref, bo1_ref, True), wo2_ref, bo2_ref, False)
    ctx = dense(dense(ctx_ref[...], wc1_ref, bc1_ref, True),
                wc2_ref, bc2_ref, False)
    z = jnp.concatenate([ha, hb, ctx], axis=1)
    z = dense(z, wm1_ref, bm1_ref, True)
    z = dense(z, wm2_ref, bm2_ref, True)
    out_ref[...] = dense(z, wm3_ref, bm3_ref, False)


_tail_kernel = pl.pallas_call(
    _tail_body,
    out_shape=jax.ShapeDtypeStruct((B, 128), jnp.float32),
)


_sc_mesh = plsc.VectorSubcoreMesh(core_axis_name="c", subcore_axis_name="s")


def _sc_aggr_body(t_hbm, comb_hbm, dst_hbm, zeros_hbm, out_hbm,
                  idxv, dstv, buf0, buf1, shared,
                  sg0, sg1, ss0, ss1):
    c = lax.axis_index("c")
    s = lax.axis_index("s")
    wid = s * 2 + c
    bufs = (buf0, buf1)
    sgs = (sg0, sg1)
    sss = (ss0, ss1)

    def gather(j, b):
        return pltpu.async_copy(t_hbm.at[idxv.at[j]], bufs[b], sgs[b])

    def scatter(j, b):
        return pltpu.async_copy(bufs[b], shared.at[dstv.at[j]], sss[b],
                                add=True)

    def wait_gather(j, b):
        pltpu.make_async_copy(t_hbm.at[idxv.at[j]], bufs[b], sgs[b]).wait()

    def wait_scatter(j, b):
        pltpu.make_async_copy(bufs[b], shared.at[dstv.at[j]], sss[b]).wait()

    pltpu.sync_copy(zeros_hbm.at[pl.ds(s * RPT, RPT)],
                    shared.at[pl.ds(s * RPT, RPT)])
    pltpu.sync_copy(comb_hbm.at[wid], idxv)
    pltpu.sync_copy(dst_hbm.at[wid], dstv)
    plsc.subcore_barrier()

    for b in range(2):
        gather(jnp.int32(b), b)

    def body(k, carry):
        j = 2 * k
        wait_gather(j, 0)
        scatter(j, 0)
        wait_gather(j + 1, 1)
        scatter(j + 1, 1)
        wait_scatter(j, 0)
        gather(jnp.minimum(j + 2, NCHUNK - 1), 0)
        wait_scatter(j + 1, 1)
        gather(jnp.minimum(j + 3, NCHUNK - 1), 1)
        return carry

    lax.fori_loop(0, NCHUNK // 2, body, jnp.int32(0))
    for b in range(2):
        wait_gather(jnp.int32(NCHUNK - 1), b)
    plsc.subcore_barrier()
    pltpu.sync_copy(shared.at[pl.ds(s * RPT, RPT)],
                    out_hbm.at[c, pl.ds(s * RPT, RPT)])


_sc_aggr = functools.partial(
    pl.kernel,
    out_type=jax.ShapeDtypeStruct((2, NODE_PAD, 128), jnp.float32),
    mesh=_sc_mesh,
    scratch_types=[
        pltpu.VMEM((NCHUNK, CHUNK), jnp.int32),
        pltpu.VMEM((NCHUNK, CHUNK), jnp.int32),
        pltpu.VMEM((CHUNK, 128), jnp.float32),
        pltpu.VMEM((CHUNK, 128), jnp.float32),
        pltpu.VMEM_SHARED((NODE_PAD, 128), jnp.float32),
        pltpu.SemaphoreType.DMA,
        pltpu.SemaphoreType.DMA,
        pltpu.SemaphoreType.DMA,
        pltpu.SemaphoreType.DMA,
    ],
)(_sc_aggr_body)


def _pad_edges(v, fill):
    return jnp.concatenate(
        [v.astype(jnp.int32), jnp.full((E_PAD - E,), fill, jnp.int32)])


def kernel(xA, edge_indexA, edge_attrA, batchA, xB, edge_indexB, edge_attrB,
           batchB, context, params):
    x_emb = params['x_emb']
    e_emb8 = jnp.concatenate(
        [params['e_emb'], jnp.zeros((2, 128), jnp.float32)], axis=0)
    zeros = jnp.zeros((NODE_PAD, 128), jnp.float32)

    def prep(x, ei, ea, batch):
        src = _pad_edges(ei[0], 0).reshape(E_PAD // 128, 128)
        eav = _pad_edges(ea[:, 0], 0).reshape(E_PAD // 128, 128)
        dst3 = _pad_edges(ei[1], N).reshape(NW, NCHUNK, CHUNK)
        comb3 = _idx_kernel(src, eav).reshape(NW, NCHUNK, CHUNK)
        x3 = x[:, 0].astype(jnp.int32).reshape(NB, 1, RB)
        batch3 = batch.astype(jnp.int32).reshape(NB, 1, RB)
        h, t = _embed_kernel(x3, x_emb, e_emb8)
        return {'comb': comb3, 'dst': dst3, 'batch': batch3, 'h': h, 't': t}

    sA = prep(xA, edge_indexA, edge_attrA, batchA)
    sB = prep(xB, edge_indexB, edge_attrB, batchB)
    for l in range(NUM_LAYER):
        p = params['gnn'][l]
        b1 = p['b1'].reshape(1, 256)
        b2 = p['b2'].reshape(1, 128)
        g = p['gamma'].reshape(1, 128)
        be = p['beta'].reshape(1, 128)
        for st_ in (sA, sB):
            st_['aggr'] = _sc_aggr(st_['t'].reshape(6 * N, 128),
                                   st_['comb'], st_['dst'], zeros)
        for st_ in (sA, sB):
            aggr = st_['aggr']
            u, st = _mlp_kernel(st_['h'], aggr[0], aggr[1],
                                p['W1'], b1, p['W2'], b2)
            if l < NUM_LAYER - 1:
                st_['h'], st_['t'] = _bn_next_kernel(u, st, g, be, e_emb8)
            else:
                st_['pooled'], st_['cnt'] = _bn_pool_kernel(
                    u, st, g, be, st_['batch'])
    pooledA, cntA = sA['pooled'], sA['cnt']
    pooledB, cntB = sB['pooled'], sB['cnt']

    ctx_pad = jnp.pad(context, ((0, 0), (0, 1024 - context.shape[1])))
    wc1_pad = jnp.pad(params['Wc1'], ((0, 1024 - params['Wc1'].shape[0]),
                                      (0, 0)))
    wm3_pad = jnp.pad(params['Wm3'], ((0, 0), (0, 127)))
    bm3_pad = jnp.pad(params['bm3'], ((0, 127),))

    out = _tail_kernel(
        pooledA, cntA, pooledB, cntB, ctx_pad,
        wc1_pad, params['bc1'].reshape(1, 512),
        params['Wc2'], params['bc2'].reshape(1, 128),
        params['Wo1'], params['bo1'].reshape(1, 512),
        params['Wo2'], params['bo2'].reshape(1, 256),
        params['Wm1'], params['bm1'].reshape(1, 256),
        params['Wm2'], params['bm2'].reshape(1, 64),
        wm3_pad, bm3_pad.reshape(1, 128))
    return out[:, :1]


# speedup vs baseline: 2.7402x; 1.0026x over previous
import functools

import jax
import jax.numpy as jnp
from jax import lax
from jax.experimental import pallas as pl
from jax.experimental.pallas import tpu as pltpu
from jax.experimental.pallas import tpu_sc as plsc

N = 10000
E = 160000
B = 64
EMB = 128
NUM_LAYER = 5

NW = 32
CHUNK = 128
EPT = 5120
E_PAD = NW * EPT
NCHUNK = EPT // CHUNK
NODE_PAD = 10112
RPT = NODE_PAD // 16
RB = 5000
NB = N // RB


def _idx_body(src_ref, ea_ref, out_ref):
    out_ref[...] = ea_ref[...] * N + src_ref[...]


_idx_kernel = pl.pallas_call(
    _idx_body,
    out_shape=jax.ShapeDtypeStruct((E_PAD // 128, 128), jnp.int32),
)


def _embed_body(x_ref, xemb_ref, eemb_ref, h_ref, t_ref):
    xb = x_ref[0, 0, :]
    oh = (xb[:, None] == lax.broadcasted_iota(jnp.int32, (RB, 120), 1))
    h = jnp.dot(oh.astype(jnp.float32), xemb_ref[...],
                preferred_element_type=jnp.float32)
    h_ref[...] = h
    for j in range(6):
        t_ref[j] = jnp.maximum(h + eemb_ref[j, :], 0.0)


_embed_kernel = pl.pallas_call(
    _embed_body,
    grid=(NB,),
    in_specs=[
        pl.BlockSpec((1, 1, RB), lambda b: (b, 0, 0)),
        pl.BlockSpec((120, 128), lambda b: (0, 0)),
        pl.BlockSpec((8, 128), lambda b: (0, 0)),
    ],
    out_specs=[
        pl.BlockSpec((RB, 128), lambda b: (b, 0)),
        pl.BlockSpec((6, RB, 128), lambda b: (0, b, 0)),
    ],
    out_shape=[
        jax.ShapeDtypeStruct((N, 128), jnp.float32),
        jax.ShapeDtypeStruct((6, N, 128), jnp.float32),
    ],
)


def _mlp_body(h_ref, a0_ref, a1_ref, w1_ref, b1_ref, w2_ref, b2_ref,
              u_ref, st_ref):
    z = h_ref[...] + a0_ref[...] + a1_ref[...]
    t = jnp.maximum(
        jnp.dot(z, w1_ref[...], preferred_element_type=jnp.float32)
        + b1_ref[0, :], 0.0)
    u = (jnp.dot(t, w2_ref[...], preferred_element_type=jnp.float32)
         + b2_ref[0, :])
    u_ref[...] = u
    s1 = jnp.sum(u, axis=0, keepdims=True)
    s2 = jnp.sum(u * u, axis=0, keepdims=True)
    st_ref[0] = jnp.concatenate([s1, s2, jnp.zeros((6, 128), jnp.float32)], 0)


_mlp_kernel = pl.pallas_call(
    _mlp_body,
    grid=(NB,),
    in_specs=[
        pl.BlockSpec((RB, 128), lambda b: (b, 0)),
        pl.BlockSpec((RB, 128), lambda b: (b, 0)),
        pl.BlockSpec((RB, 128), lambda b: (b, 0)),
        pl.BlockSpec((128, 256), lambda b: (0, 0)),
        pl.BlockSpec((1, 256), lambda b: (0, 0)),
        pl.BlockSpec((256, 128), lambda b: (0, 0)),
        pl.BlockSpec((1, 128), lambda b: (0, 0)),
    ],
    out_specs=[
        pl.BlockSpec((RB, 128), lambda b: (b, 0)),
        pl.BlockSpec((1, 8, 128), lambda b: (b, 0, 0)),
    ],
    out_shape=[
        jax.ShapeDtypeStruct((N, 128), jnp.float32),
        jax.ShapeDtypeStruct((NB, 8, 128), jnp.float32),
    ],
)


def _bn_scale_shift(st_ref, g_ref, be_ref):
    s = st_ref[...]
    mu = jnp.sum(s[:, 0, :], axis=0) / N
    ex2 = jnp.sum(s[:, 1, :], axis=0) / N
    var = ex2 - mu * mu
    scale = g_ref[0, :] * lax.rsqrt(var + 1e-5)
    shift = be_ref[0, :] - mu * scale
    return scale, shift


def _bn_next_body(u_ref, st_ref, g_ref, be_ref, eemb_ref, h_ref, t_ref):
    scale, shift = _bn_scale_shift(st_ref, g_ref, be_ref)
    hn = jnp.maximum(u_ref[...] * scale + shift, 0.0)
    h_ref[...] = hn
    for j in range(6):
        t_ref[j] = jnp.maximum(hn + eemb_ref[j, :], 0.0)


_bn_next_kernel = pl.pallas_call(
    _bn_next_body,
    grid=(NB,),
    in_specs=[
        pl.BlockSpec((RB, 128), lambda b: (b, 0)),
        pl.BlockSpec((NB, 8, 128), lambda b: (0, 0, 0)),
        pl.BlockSpec((1, 128), lambda b: (0, 0)),
        pl.BlockSpec((1, 128), lambda b: (0, 0)),
        pl.BlockSpec((8, 128), lambda b: (0, 0)),
    ],
    out_specs=[
        pl.BlockSpec((RB, 128), lambda b: (b, 0)),
        pl.BlockSpec((6, RB, 128), lambda b: (0, b, 0)),
    ],
    out_shape=[
        jax.ShapeDtypeStruct((N, 128), jnp.float32),
        jax.ShapeDtypeStruct((6, N, 128), jnp.float32),
    ],
)


def _bn_pool_body(u_ref, st_ref, g_ref, be_ref, batch_ref,
                  pooled_ref, cnt_ref):
    scale, shift = _bn_scale_shift(st_ref, g_ref, be_ref)
    hn = u_ref[...] * scale + shift
    b = pl.program_id(0)

    @pl.when(b == 0)
    def _():
        pooled_ref[...] = jnp.zeros((B, 128), jnp.float32)
        cnt_ref[...] = jnp.zeros((B, 128), jnp.float32)

    bb = batch_ref[0, 0, :]
    mask = (lax.broadcasted_iota(jnp.int32, (B, RB), 0)
            == bb[None, :]).astype(jnp.float32)
    pooled_ref[...] += jnp.dot(mask, hn, preferred_element_type=jnp.float32)
    cnt_ref[...] += jnp.broadcast_to(jnp.sum(mask, axis=1)[:, None], (B, 128))


_bn_pool_kernel = pl.pallas_call(
    _bn_pool_body,
    grid=(NB,),
    in_specs=[
        pl.BlockSpec((RB, 128), lambda b: (b, 0)),
        pl.BlockSpec((NB, 8, 128), lambda b: (0, 0, 0)),
        pl.BlockSpec((1, 128), lambda b: (0, 0)),
        pl.BlockSpec((1, 128), lambda b: (0, 0)),
        pl.BlockSpec((1, 1, RB), lambda b: (b, 0, 0)),
    ],
    out_specs=[
        pl.BlockSpec((B, 128), lambda b: (0, 0)),
        pl.BlockSpec((B, 128), lambda b: (0, 0)),
    ],
    out_shape=[
        jax.ShapeDtypeStruct((B, 128), jnp.float32),
        jax.ShapeDtypeStruct((B, 128), jnp.float32),
    ],
)


def _tail_body(pa_ref, ca_ref, pb_ref, cb_ref, ctx_ref,
               wc1_ref, bc1_ref, wc2_ref, bc2_ref,
               wo1_ref, bo1_ref, wo2_ref, bo2_ref,
               wm1_ref, bm1_ref, wm2_ref, bm2_ref, wm3_ref, bm3_ref,
               out_ref):
    def dense(x, w_ref, b_ref, act):
        y = (jnp.dot(x, w_ref[...], preferred_element_type=jnp.float32)
             + b_ref[0, :])
        return jnp.maximum(y, 0.0) if act else y

    pa = pa_ref[...] / jnp.maximum(ca_ref[...], 1.0)
    pb = pb_ref[...] / jnp.maximum(cb_ref[...], 1.0)
    ha = dense(dense(pa, wo1_ref, bo1_ref, True), wo2_ref, bo2_ref, False)
    hb = dense(dense(pb, wo1_ref, bo1_ref, True), wo2_ref, bo2_ref, False)
    ctx = dense(dense(ctx_ref[...], wc1_ref, bc1_ref, True),
                wc2_ref, bc2_ref, False)
    z = jnp.concatenate([ha, hb, ctx], axis=1)
    z = dense(z, wm1_ref, bm1_ref, True)
    z = dense(z, wm2_ref, bm2_ref, True)
    out_ref[...] = dense(z, wm3_ref, bm3_ref, False)


_tail_kernel = pl.pallas_call(
    _tail_body,
    out_shape=jax.ShapeDtypeStruct((B, 128), jnp.float32),
)


_sc_mesh = plsc.VectorSubcoreMesh(core_axis_name="c", subcore_axis_name="s")


def _sc_aggr_body(t_hbm, comb_hbm, dst_hbm, zeros_hbm, out_hbm,
                  idxv, dstv, buf0, buf1, shared,
                  sg0, sg1, ss0, ss1):
    c = lax.axis_index("c")
    s = lax.axis_index("s")
    wid = s * 2 + c
    bufs = (buf0, buf1)
    sgs = (sg0, sg1)
    sss = (ss0, ss1)

    def gather(j, b):
        return pltpu.async_copy(t_hbm.at[idxv.at[j]], bufs[b], sgs[b])

    def scatter(j, b):
        return pltpu.async_copy(bufs[b], shared.at[dstv.at[j]], sss[b],
                                add=True)

    def wait_gather(j, b):
        pltpu.make_async_copy(t_hbm.at[idxv.at[j]], bufs[b], sgs[b]).wait()

    def wait_scatter(j, b):
        pltpu.make_async_copy(bufs[b], shared.at[dstv.at[j]], sss[b]).wait()

    pltpu.sync_copy(zeros_hbm.at[pl.ds(s * RPT, RPT)],
                    shared.at[pl.ds(s * RPT, RPT)])
    pltpu.sync_copy(comb_hbm.at[wid], idxv)
    pltpu.sync_copy(dst_hbm.at[wid], dstv)
    plsc.subcore_barrier()

    for b in range(2):
        gather(jnp.int32(b), b)

    def body(k, carry):
        j = 2 * k
        wait_gather(j, 0)
        scatter(j, 0)
        wait_gather(j + 1, 1)
        scatter(j + 1, 1)
        wait_scatter(j, 0)
        gather(jnp.minimum(j + 2, NCHUNK - 1), 0)
        wait_scatter(j + 1, 1)
        gather(jnp.minimum(j + 3, NCHUNK - 1), 1)
        return carry

    lax.fori_loop(0, NCHUNK // 2, body, jnp.int32(0))
    for b in range(2):
        wait_gather(jnp.int32(NCHUNK - 1), b)
    plsc.subcore_barrier()
    pltpu.sync_copy(shared.at[pl.ds(s * RPT, RPT)],
                    out_hbm.at[c, pl.ds(s * RPT, RPT)])


_sc_aggr = functools.partial(
    pl.kernel,
    out_type=jax.ShapeDtypeStruct((2, NODE_PAD, 128), jnp.float32),
    mesh=_sc_mesh,
    scratch_types=[
        pltpu.VMEM((NCHUNK, CHUNK), jnp.int32),
        pltpu.VMEM((NCHUNK, CHUNK), jnp.int32),
        pltpu.VMEM((CHUNK, 128), jnp.float32),
        pltpu.VMEM((CHUNK, 128), jnp.float32),
        pltpu.VMEM_SHARED((NODE_PAD, 128), jnp.float32),
        pltpu.SemaphoreType.DMA,
        pltpu.SemaphoreType.DMA,
        pltpu.SemaphoreType.DMA,
        pltpu.SemaphoreType.DMA,
    ],
)(_sc_aggr_body)


def _pad_edges(v, fill):
    return jnp.concatenate(
        [v.astype(jnp.int32), jnp.full((E_PAD - E,), fill, jnp.int32)])


def kernel(xA, edge_indexA, edge_attrA, batchA, xB, edge_indexB, edge_attrB,
           batchB, context, params):
    x_emb = params['x_emb']
    e_emb8 = jnp.concatenate(
        [params['e_emb'], jnp.zeros((2, 128), jnp.float32)], axis=0)
    zeros = jnp.zeros((NODE_PAD, 128), jnp.float32)

    def prep(x, ei, ea, batch):
        src = _pad_edges(ei[0], 0).reshape(E_PAD // 128, 128)
        eav = _pad_edges(ea[:, 0], 0).reshape(E_PAD // 128, 128)
        dst3 = _pad_edges(ei[1], N).reshape(NW, NCHUNK, CHUNK)
        comb3 = _idx_kernel(src, eav).reshape(NW, NCHUNK, CHUNK)
        x3 = x[:, 0].astype(jnp.int32).reshape(NB, 1, RB)
        batch3 = batch.astype(jnp.int32).reshape(NB, 1, RB)
        h, t = _embed_kernel(x3, x_emb, e_emb8)
        return {'comb': comb3, 'dst': dst3, 'batch': batch3, 'h': h, 't': t}

    sA = prep(xA, edge_indexA, edge_attrA, batchA)
    sB = prep(xB, edge_indexB, edge_attrB, batchB)
    for l in range(NUM_LAYER):
        p = params['gnn'][l]
        b1 = p['b1'].reshape(1, 256)
        b2 = p['b2'].reshape(1, 128)
        g = p['gamma'].reshape(1, 128)
        be = p['beta'].reshape(1, 128)
        for st_ in (sA, sB):
            st_['aggr'] = _sc_aggr(st_['t'].reshape(6 * N, 128),
                                   st_['comb'], st_['dst'], zeros)
        for st_ in (sA, sB):
            aggr = st_['aggr']
            u, st = _mlp_kernel(st_['h'], aggr[0], aggr[1],
                                p['W1'], b1, p['W2'], b2)
            if l < NUM_LAYER - 1:
                st_['h'], st_['t'] = _bn_next_kernel(u, st, g, be, e_emb8)
            else:
                st_['pooled'], st_['cnt'] = _bn_pool_kernel(
                    u, st, g, be, st_['batch'])
    pooledA, cntA = sA['pooled'], sA['cnt']
    pooledB, cntB = sB['pooled'], sB['cnt']

    ctx_pad = jnp.pad(context, ((0, 0), (0, 1024 - context.shape[1])))
    wc1_pad = jnp.pad(params['Wc1'], ((0, 1024 - params['Wc1'].shape[0]),
                                      (0, 0)))
    wm3_pad = jnp.pad(params['Wm3'], ((0, 0), (0, 127)))
    bm3_pad = jnp.pad(params['bm3'], ((0, 127),))

    out = _tail_kernel(
        pooledA, cntA, pooledB, cntB, ctx_pad,
        wc1_pad, params['bc1'].reshape(1, 512),
        params['Wc2'], params['bc2'].reshape(1, 128),
        params['Wo1'], params['bo1'].reshape(1, 512),
        params['Wo2'], params['bo2'].reshape(1, 256),
        params['Wm1'], params['bm1'].reshape(1, 256),
        params['Wm2'], params['bm2'].reshape(1, 64),
        wm3_pad, bm3_pad.reshape(1, 128))
    return out[:, :1]
